# baseline (speedup 1.0000x reference)
import jax
import jax.numpy as jnp
from jax.experimental import pallas as pl
from jax.experimental.pallas import tpu as pltpu

EPS = 1e-5


def _im2col_matmul(pad_ref, slab_ref, w_ref, hp, wp, cin):
    for dh in range(3):
        for dw in range(3):
            t = dh * 3 + dw
            slab_ref[:, :, t * cin:(t + 1) * cin] = (
                pad_ref[dh:dh + hp, dw:dw + wp, :])
    return jnp.dot(slab_ref[...].reshape(hp * wp, 9 * cin), w_ref[...],
                   preferred_element_type=jnp.float32)


def _write_stats(ssq_ref, acc):
    s = jnp.sum(acc, axis=0, keepdims=True)
    sq = jnp.sum(acc * acc, axis=0, keepdims=True)
    ssq_ref[0] = jnp.concatenate([s, sq], axis=0)


def _pool_conv1_kernel(x_ref, w_ref, y_ref, ssq_ref, xs_ref, pad_ref,
                       slab_ref):
    cin, hw = x_ref.shape[1], x_ref.shape[2]
    H, W, _ = xs_ref.shape
    hp, wp = H // 2, W // 2
    xs_ref[...] = jnp.transpose(x_ref[0]).reshape(H, W, cin)
    a = xs_ref[:, pl.ds(0, wp, stride=2), :]
    b = xs_ref[:, pl.ds(1, wp, stride=2), :]
    xw = jnp.maximum(a, b)
    pooled = jnp.max(xw.reshape(hp, 2, wp, cin), axis=1).astype(jnp.bfloat16)

    zrow = jnp.zeros((1, wp + 2, cin), jnp.bfloat16)
    zcol = jnp.zeros((hp, 1, cin), jnp.bfloat16)
    pad_ref[0:1, :, :] = zrow
    pad_ref[hp + 1:hp + 2, :, :] = zrow
    pad_ref[1:hp + 1, 0:1, :] = zcol
    pad_ref[1:hp + 1, wp + 1:wp + 2, :] = zcol
    pad_ref[1:hp + 1, 1:wp + 1, :] = pooled

    acc = _im2col_matmul(pad_ref, slab_ref, w_ref, hp, wp, cin)
    y_ref[0] = acc.reshape(hp, wp, -1).astype(jnp.bfloat16)
    _write_stats(ssq_ref, acc)


def _bn_relu_conv2_kernel(y1_ref, sc_ref, sh_ref, w_ref, y_ref, ssq_ref,
                          pad_ref, slab_ref):
    hp, wp, c = y1_ref.shape[1], y1_ref.shape[2], y1_ref.shape[3]
    h = jnp.maximum(
        y1_ref[0].astype(jnp.float32) * sc_ref[0] + sh_ref[0], 0.0)
    hb = h.astype(jnp.bfloat16)

    zrow = jnp.zeros((1, wp + 2, c), jnp.bfloat16)
    zcol = jnp.zeros((hp, 1, c), jnp.bfloat16)
    pad_ref[0:1, :, :] = zrow
    pad_ref[hp + 1:hp + 2, :, :] = zrow
    pad_ref[1:hp + 1, 0:1, :] = zcol
    pad_ref[1:hp + 1, wp + 1:wp + 2, :] = zcol
    pad_ref[1:hp + 1, 1:wp + 1, :] = hb

    acc = _im2col_matmul(pad_ref, slab_ref, w_ref, hp, wp, c)
    y_ref[0] = acc.reshape(hp, wp, -1).astype(jnp.bfloat16)
    _write_stats(ssq_ref, acc)


def _bn_relu_out_kernel(y2_ref, sc_ref, sh_ref, out_ref):
    hp, wp, c = y2_ref.shape[1], y2_ref.shape[2], y2_ref.shape[3]
    h = jnp.maximum(
        y2_ref[0].astype(jnp.float32) * sc_ref[0] + sh_ref[0], 0.0)
    out_ref[0] = jnp.transpose(h.reshape(hp * wp, c))


def _pack_conv_w(w_oihw):
    w = jnp.transpose(w_oihw, (2, 3, 1, 0))
    co = w_oihw.shape[0]
    ci = w_oihw.shape[1]
    return w.reshape(9 * ci, co).astype(jnp.bfloat16)


def _finalize_bn(stats, gamma, beta, count):
    s = jnp.sum(stats[:, 0, :], axis=0)
    sq = jnp.sum(stats[:, 1, :], axis=0)
    mean = s / count
    var = jnp.maximum(sq / count - mean * mean, 0.0)
    scale = gamma * jax.lax.rsqrt(var + EPS)
    shift = beta - mean * scale
    c = gamma.shape[0]
    return scale.reshape(1, c), shift.reshape(1, c)


@jax.jit
def kernel(x_nchw, w1, b1, g1, be1, w2, b2, g2, be2):
    del b1, b2
    N, Cin, H, W = x_nchw.shape
    Hp, Wp = H // 2, W // 2
    Cout = w1.shape[0]

    x_flat = x_nchw.reshape(N, Cin, H * W)
    w1p = _pack_conv_w(w1)
    w2p = _pack_conv_w(w2)
    count = float(N * Hp * Wp)

    cparams = pltpu.CompilerParams(
        dimension_semantics=("parallel",),
        vmem_limit_bytes=64 * 1024 * 1024)

    y1, st1 = pl.pallas_call(
        _pool_conv1_kernel,
        grid=(N,),
        in_specs=[
            pl.BlockSpec((1, Cin, H * W), lambda i: (i, 0, 0)),
            pl.BlockSpec((9 * Cin, Cout), lambda i: (0, 0)),
        ],
        out_specs=(
            pl.BlockSpec((1, Hp, Wp, Cout), lambda i: (i, 0, 0, 0)),
            pl.BlockSpec((1, 2, Cout), lambda i: (i, 0, 0)),
        ),
        out_shape=(
            jax.ShapeDtypeStruct((N, Hp, Wp, Cout), jnp.bfloat16),
            jax.ShapeDtypeStruct((N, 2, Cout), jnp.float32),
        ),
        scratch_shapes=[
            pltpu.VMEM((H, W, Cin), jnp.float32),
            pltpu.VMEM((Hp + 2, Wp + 2, Cin), jnp.bfloat16),
            pltpu.VMEM((Hp, Wp, 9 * Cin), jnp.bfloat16),
        ],
        compiler_params=cparams,
    )(x_flat, w1p)

    sc1, sh1 = _finalize_bn(st1, g1, be1, count)

    y2, st2 = pl.pallas_call(
        _bn_relu_conv2_kernel,
        grid=(N,),
        in_specs=[
            pl.BlockSpec((1, Hp, Wp, Cout), lambda i: (i, 0, 0, 0)),
            pl.BlockSpec((1, Cout), lambda i: (0, 0)),
            pl.BlockSpec((1, Cout), lambda i: (0, 0)),
            pl.BlockSpec((9 * Cout, Cout), lambda i: (0, 0)),
        ],
        out_specs=(
            pl.BlockSpec((1, Hp, Wp, Cout), lambda i: (i, 0, 0, 0)),
            pl.BlockSpec((1, 2, Cout), lambda i: (i, 0, 0)),
        ),
        out_shape=(
            jax.ShapeDtypeStruct((N, Hp, Wp, Cout), jnp.bfloat16),
            jax.ShapeDtypeStruct((N, 2, Cout), jnp.float32),
        ),
        scratch_shapes=[
            pltpu.VMEM((Hp + 2, Wp + 2, Cout), jnp.bfloat16),
            pltpu.VMEM((Hp, Wp, 9 * Cout), jnp.bfloat16),
        ],
        compiler_params=cparams,
    )(y1, sc1, sh1, w2p)

    sc2, sh2 = _finalize_bn(st2, g2, be2, count)

    out = pl.pallas_call(
        _bn_relu_out_kernel,
        grid=(N,),
        in_specs=[
            pl.BlockSpec((1, Hp, Wp, Cout), lambda i: (i, 0, 0, 0)),
            pl.BlockSpec((1, Cout), lambda i: (0, 0)),
            pl.BlockSpec((1, Cout), lambda i: (0, 0)),
        ],
        out_specs=pl.BlockSpec((1, Cout, Hp * Wp), lambda i: (i, 0, 0)),
        out_shape=jax.ShapeDtypeStruct((N, Cout, Hp * Wp), jnp.float32),
        compiler_params=cparams,
    )(y2, sc2, sh2)

    return out.reshape(N, Cout, Hp, Wp)


# speedup vs baseline: 1.1085x; 1.1085x over previous
import functools

import jax
import jax.numpy as jnp
from jax.experimental import pallas as pl
from jax.experimental.pallas import tpu as pltpu

EPS = 1e-5
LANE = 128


def _build_slab_and_matmul(pad_ref, slab_ref, w_ref, hp, wp):
    for dh in range(3):
        for dw in range(3):
            t = dh * 3 + dw
            slab_ref[:, :, t * LANE:(t + 1) * LANE] = (
                pad_ref[dh:dh + hp, dw:dw + wp, :])
    return jnp.dot(slab_ref[...].reshape(hp * wp, 9 * LANE), w_ref[...],
                   preferred_element_type=jnp.float32)


def _halo_zero(pad_ref, hp, wp):
    zrow = jnp.zeros((1, wp + 2, LANE), jnp.bfloat16)
    zcol = jnp.zeros((hp, 1, LANE), jnp.bfloat16)
    pad_ref[0:1, :, :] = zrow
    pad_ref[hp + 1:hp + 2, :, :] = zrow
    pad_ref[1:hp + 1, 0:1, :] = zcol
    pad_ref[1:hp + 1, wp + 1:wp + 2, :] = zcol


def _scale_shift(st_ref, g_ref, b_ref, count):
    s = st_ref[0, :]
    sq = st_ref[1, :]
    mean = s / count
    var = jnp.maximum(sq / count - mean * mean, 0.0)
    scale = g_ref[0] * jax.lax.rsqrt(var + EPS)
    shift = b_ref[0] - mean * scale
    return scale, shift


def _down_kernel(count, x_ref, w1_ref, w2_ref, g1_ref, b1_ref, g2_ref,
                 b2_ref, out_ref, xs_ref, y_ref, pad_ref, slab_ref,
                 st1_ref, st2_ref):
    p = pl.program_id(0)
    i = pl.program_id(1)
    cin = x_ref.shape[1]
    H = xs_ref.shape[0]
    W = xs_ref.shape[1]
    hp, wp = H // 2, W // 2

    @pl.when(p == 0)
    def _phase0():
        @pl.when(i == 0)
        def _():
            st1_ref[...] = jnp.zeros_like(st1_ref)

        xs_ref[...] = jnp.transpose(x_ref[0]).reshape(H, W, cin)
        a = xs_ref[:, pl.ds(0, wp, stride=2), :]
        b = xs_ref[:, pl.ds(1, wp, stride=2), :]
        xw = jnp.maximum(a, b)
        pooled = jnp.max(xw.reshape(hp, 2, wp, cin), axis=1)
        pooled = jnp.pad(pooled.astype(jnp.bfloat16),
                         ((0, 0), (0, 0), (0, LANE - cin)))

        _halo_zero(pad_ref, hp, wp)
        pad_ref[1:hp + 1, 1:wp + 1, :] = pooled
        acc = _build_slab_and_matmul(pad_ref, slab_ref, w1_ref, hp, wp)
        y_ref[i] = acc.reshape(hp, wp, LANE).astype(jnp.bfloat16)
        s = jnp.sum(acc, axis=0, keepdims=True)
        sq = jnp.sum(acc * acc, axis=0, keepdims=True)
        st1_ref[...] += jnp.concatenate([s, sq], axis=0)

    @pl.when(p == 1)
    def _phase1():
        @pl.when(i == 0)
        def _():
            st2_ref[...] = jnp.zeros_like(st2_ref)

        scale, shift = _scale_shift(st1_ref, g1_ref, b1_ref, count)
        h = jnp.maximum(y_ref[i].astype(jnp.float32) * scale + shift, 0.0)

        _halo_zero(pad_ref, hp, wp)
        pad_ref[1:hp + 1, 1:wp + 1, :] = h.astype(jnp.bfloat16)
        acc = _build_slab_and_matmul(pad_ref, slab_ref, w2_ref, hp, wp)
        y_ref[i] = acc.reshape(hp, wp, LANE).astype(jnp.bfloat16)
        s = jnp.sum(acc, axis=0, keepdims=True)
        sq = jnp.sum(acc * acc, axis=0, keepdims=True)
        st2_ref[...] += jnp.concatenate([s, sq], axis=0)

    @pl.when(p == 2)
    def _phase2():
        scale, shift = _scale_shift(st2_ref, g2_ref, b2_ref, count)
        h = jnp.maximum(y_ref[i].astype(jnp.float32) * scale + shift, 0.0)
        out_ref[0] = jnp.transpose(h.reshape(hp * wp, LANE))


def _pack_conv_w(w_oihw):
    co, ci, _, _ = w_oihw.shape
    w = jnp.transpose(w_oihw, (2, 3, 1, 0))
    w = jnp.pad(w, ((0, 0), (0, 0), (0, LANE - ci), (0, 0)))
    return w.reshape(9 * LANE, co).astype(jnp.bfloat16)


@jax.jit
def kernel(x_nchw, w1, b1, g1, be1, w2, b2, g2, be2):
    del b1, b2
    N, Cin, H, W = x_nchw.shape
    Hp, Wp = H // 2, W // 2
    Cout = w1.shape[0]

    x_flat = x_nchw.reshape(N, Cin, H * W)
    w1p = _pack_conv_w(w1)
    w2p = _pack_conv_w(w2)
    count = float(N * Hp * Wp)

    body = functools.partial(_down_kernel, count)

    out = pl.pallas_call(
        body,
        grid=(3, N),
        in_specs=[
            pl.BlockSpec((1, Cin, H * W), lambda p, i: ((p == 0) * i, 0, 0)),
            pl.BlockSpec((9 * LANE, Cout), lambda p, i: (0, 0)),
            pl.BlockSpec((9 * LANE, Cout), lambda p, i: (0, 0)),
            pl.BlockSpec((1, Cout), lambda p, i: (0, 0)),
            pl.BlockSpec((1, Cout), lambda p, i: (0, 0)),
            pl.BlockSpec((1, Cout), lambda p, i: (0, 0)),
            pl.BlockSpec((1, Cout), lambda p, i: (0, 0)),
        ],
        out_specs=pl.BlockSpec((1, Cout, Hp * Wp),
                               lambda p, i: ((p == 2) * i, 0, 0)),
        out_shape=jax.ShapeDtypeStruct((N, Cout, Hp * Wp), jnp.float32),
        scratch_shapes=[
            pltpu.VMEM((H, W, Cin), jnp.float32),
            pltpu.VMEM((N, Hp, Wp, LANE), jnp.bfloat16),
            pltpu.VMEM((Hp + 2, Wp + 2, LANE), jnp.bfloat16),
            pltpu.VMEM((Hp, Wp, 9 * LANE), jnp.bfloat16),
            pltpu.VMEM((2, LANE), jnp.float32),
            pltpu.VMEM((2, LANE), jnp.float32),
        ],
        compiler_params=pltpu.CompilerParams(
            dimension_semantics=("arbitrary", "arbitrary"),
            vmem_limit_bytes=60 * 1024 * 1024),
    )(x_flat, w1p, w2p, g1.reshape(1, Cout), be1.reshape(1, Cout),
      g2.reshape(1, Cout), be2.reshape(1, Cout))

    return out.reshape(N, Cout, Hp, Wp)
